# baseline (device time: 99402 ns/iter reference)
import jax

jax.config.update("jax_compilation_cache_dir", "/tmp/scband_jax_cache")
jax.config.update("jax_persistent_cache_min_compile_time_secs", 0.0)
jax.config.update("jax_persistent_cache_min_entry_size_bytes", 0)

import jax.numpy as jnp
from jax import lax
from jax.experimental import pallas as pl
from jax.experimental.pallas import tpu as pltpu

N_DEV = 4
SQ = 1024
SKV = 1024
D_MODEL = 1024
HQ_PER = 8
HH = HQ_PER // 2
DH = 128
HALF = HH * DH
SCALE = 0.08838834764831843
BLK = 64
TS = 256
T = SQ // TS


def kernel(x, Wq, K_ext, V_ext, Wo):
    x16 = x.astype(jnp.bfloat16)
    wq16 = Wq.astype(jnp.bfloat16)
    wo16 = Wo.astype(jnp.bfloat16)

    def body(x_ref, wq_ref, k_hbm, v_hbm, wo_ref, out_ref,
             cwq, cwo, wwq, wwo,
             kcw, vcw, kww, vww, ctx3, mask_buf,
             ssem, rsem, kv_sems):
        my = lax.axis_index("i")
        right = lax.rem(my + 1, N_DEV)
        left = lax.rem(my + 3, N_DEV)

        def kv_copies(s, p):
            c_cw = lax.rem(my - s + N_DEV, N_DEV)
            c_ww = lax.rem(my + s, N_DEV)
            copies = []
            for i in range(HH):
                copies += [
                    pltpu.make_async_copy(
                        k_hbm.at[my, :, c_cw * HQ_PER + i, :],
                        kcw.at[p, i], kv_sems.at[p, i]),
                    pltpu.make_async_copy(
                        v_hbm.at[my, :, c_cw * HQ_PER + i, :],
                        vcw.at[p, i], kv_sems.at[p, HH + i]),
                    pltpu.make_async_copy(
                        k_hbm.at[my, :, c_ww * HQ_PER + HH + i, :],
                        kww.at[p, i], kv_sems.at[p, 2 * HH + i]),
                    pltpu.make_async_copy(
                        v_hbm.at[my, :, c_ww * HQ_PER + HH + i, :],
                        vww.at[p, i], kv_sems.at[p, 3 * HH + i]),
                ]
            return copies

        for cp in kv_copies(0, 0):
            cp.start()

        cwq[0] = wq_ref[:, :HALF]
        cwo[0] = wo_ref[:HALF, :]
        wwq[0] = wq_ref[:, HALF:]
        wwo[0] = wo_ref[HALF:, :]

        rb = lax.broadcasted_iota(jnp.int32, (TS, TS), 0) // BLK
        cb = lax.broadcasted_iota(jnp.int32, (TS, TS), 1) // BLK
        mask_buf[...] = jnp.where(cb <= rb, 0.0, -1e9)

        out_ref[0] = jnp.zeros((SQ, D_MODEL), jnp.float32)
        xb = x_ref[0]

        barrier = pltpu.get_barrier_semaphore()
        for nbr in (left, right):
            pl.semaphore_signal(
                barrier, inc=1,
                device_id=(nbr,), device_id_type=pl.DeviceIdType.MESH,
            )
        pl.semaphore_wait(barrier, 2)

        def chain(buf, ci, hh, dev):
            return pltpu.make_async_remote_copy(
                src_ref=buf.at[hh], dst_ref=buf.at[hh + 1],
                send_sem=ssem.at[ci, hh], recv_sem=rsem.at[ci, hh],
                device_id=(dev,), device_id_type=pl.DeviceIdType.MESH)

        def step(h, carry):
            par = lax.rem(h, 2)
            hm = lax.max(h - 1, 0)
            hc = lax.min(h, N_DEV - 2)

            @pl.when(h > 0)
            def _():
                chain(cwq, 0, hm, right).wait_recv()
                chain(wwq, 2, hm, left).wait_recv()

            @pl.when(h < N_DEV - 1)
            def _():
                chain(cwq, 0, hc, right).start()
                chain(wwq, 2, hc, left).start()

            @pl.when(h < N_DEV - 1)
            def _():
                for cp in kv_copies(h + 1, 1 - par):
                    cp.start()

            for cp in kv_copies(h, par):
                cp.wait()

            ck = (((1,), (1,)), ((), ()))
            cv = (((1,), (0,)), ((), ()))
            for qbuf, obuf, kbuf, vbuf, oci, dev in (
                (cwq, cwo, kcw, vcw, 1, right),
                (wwq, wwo, kww, vww, 3, left),
            ):
                wq_h = qbuf[h]
                q = lax.dot_general(
                    xb, wq_h, (((1,), (0,)), ((), ())),
                    preferred_element_type=jnp.float32)
                for i in range(HH):
                    for t in range(T):
                        r0 = t * TS
                        qt = q[r0:r0 + TS, i * DH:(i + 1) * DH]
                        sd = lax.dot_general(
                            qt, kbuf[par, i, r0:r0 + TS, :], ck,
                            preferred_element_type=jnp.float32)
                        wd = jnp.exp(sd * SCALE + mask_buf[...])
                        den = jnp.sum(wd, axis=1, keepdims=True)
                        ce = lax.dot_general(
                            wd, vbuf[par, i, r0:r0 + TS, :], cv,
                            preferred_element_type=jnp.float32)
                        if t > 0:
                            sf = lax.dot_general(
                                qt, kbuf[par, i, :r0, :], ck,
                                preferred_element_type=jnp.float32)
                            wf = jnp.exp(sf * SCALE)
                            den = den + jnp.sum(wf, axis=1, keepdims=True)
                            ce = ce + lax.dot_general(
                                wf, vbuf[par, i, :r0, :], cv,
                                preferred_element_type=jnp.float32)
                        ctx3[r0:r0 + TS, i * DH:(i + 1) * DH] = ce / den
                ctxt = ctx3[...].astype(jnp.bfloat16)

                @pl.when(h > 0)
                def _():
                    chain(obuf, oci, hm, dev).wait_recv()

                @pl.when(h < N_DEV - 1)
                def _():
                    chain(obuf, oci, hc, dev).start()

                contrib = lax.dot_general(
                    ctxt, obuf[h], (((1,), (0,)), ((), ())),
                    preferred_element_type=jnp.float32)
                out_ref[0] = out_ref[0] + contrib

            return carry

        lax.fori_loop(0, N_DEV, step, 0)

        for hh in range(N_DEV - 1):
            chain(cwq, 0, hh, right).wait_send()
            chain(cwo, 1, hh, right).wait_send()
            chain(wwq, 2, hh, left).wait_send()
            chain(wwo, 3, hh, left).wait_send()

    return pl.pallas_call(
        body,
        out_shape=jax.ShapeDtypeStruct((1, SQ, D_MODEL), jnp.float32),
        in_specs=[
            pl.BlockSpec(memory_space=pltpu.VMEM),
            pl.BlockSpec(memory_space=pltpu.VMEM),
            pl.BlockSpec(memory_space=pl.ANY),
            pl.BlockSpec(memory_space=pl.ANY),
            pl.BlockSpec(memory_space=pltpu.VMEM),
        ],
        out_specs=pl.BlockSpec(memory_space=pltpu.VMEM),
        scratch_shapes=[
            pltpu.VMEM((N_DEV, D_MODEL, HALF), jnp.bfloat16),
            pltpu.VMEM((N_DEV, HALF, D_MODEL), jnp.bfloat16),
            pltpu.VMEM((N_DEV, D_MODEL, HALF), jnp.bfloat16),
            pltpu.VMEM((N_DEV, HALF, D_MODEL), jnp.bfloat16),
            pltpu.VMEM((2, HH, SKV, DH), jnp.float32),
            pltpu.VMEM((2, HH, SKV, DH), jnp.float32),
            pltpu.VMEM((2, HH, SKV, DH), jnp.float32),
            pltpu.VMEM((2, HH, SKV, DH), jnp.float32),
            pltpu.VMEM((SQ, HALF), jnp.float32),
            pltpu.VMEM((TS, TS), jnp.float32),
            pltpu.SemaphoreType.DMA((4, N_DEV - 1)),
            pltpu.SemaphoreType.DMA((4, N_DEV - 1)),
            pltpu.SemaphoreType.DMA((2, 4 * HH)),
        ],
        compiler_params=pltpu.CompilerParams(
            collective_id=0,
            vmem_limit_bytes=58 * 1024 * 1024,
        ),
    )(x16, wq16, K_ext, V_ext, wo16)


# device time: 92680 ns/iter; 1.0725x vs baseline; 1.0725x over previous
import jax

jax.config.update("jax_compilation_cache_dir", "/tmp/scband_jax_cache")
jax.config.update("jax_persistent_cache_min_compile_time_secs", 0.0)
jax.config.update("jax_persistent_cache_min_entry_size_bytes", 0)

import jax.numpy as jnp
from jax import lax
from jax.experimental import pallas as pl
from jax.experimental.pallas import tpu as pltpu

N_DEV = 4
SQ = 1024
SKV = 1024
D_MODEL = 1024
HQ_PER = 8
HH = HQ_PER // 2
DH = 128
HALF = HH * DH
SCALE = 0.08838834764831843
BLK = 64
TS = 256
T = SQ // TS


def kernel(x, Wq, K_ext, V_ext, Wo):
    def body(x_ref, wq_ref, k_hbm, v_hbm, wo_ref, out_ref,
             cwq, cwo, wwq, wwo,
             kcw, vcw, kww, vww, ctx3, mask_buf,
             ssem, rsem, kv_sems):
        my = lax.axis_index("i")
        right = lax.rem(my + 1, N_DEV)
        left = lax.rem(my + 3, N_DEV)

        def kv_copies(s, p):
            c_cw = lax.rem(my - s + N_DEV, N_DEV)
            c_ww = lax.rem(my + s, N_DEV)
            copies = []
            for i in range(HH):
                copies += [
                    pltpu.make_async_copy(
                        k_hbm.at[my, :, c_cw * HQ_PER + i, :],
                        kcw.at[p, i], kv_sems.at[p, i]),
                    pltpu.make_async_copy(
                        v_hbm.at[my, :, c_cw * HQ_PER + i, :],
                        vcw.at[p, i], kv_sems.at[p, HH + i]),
                    pltpu.make_async_copy(
                        k_hbm.at[my, :, c_ww * HQ_PER + HH + i, :],
                        kww.at[p, i], kv_sems.at[p, 2 * HH + i]),
                    pltpu.make_async_copy(
                        v_hbm.at[my, :, c_ww * HQ_PER + HH + i, :],
                        vww.at[p, i], kv_sems.at[p, 3 * HH + i]),
                ]
            return copies

        for cp in kv_copies(0, 0):
            cp.start()

        cwq[0] = wq_ref[:, :HALF].astype(jnp.bfloat16)
        cwo[0] = wo_ref[:HALF, :].astype(jnp.bfloat16)
        wwq[0] = wq_ref[:, HALF:].astype(jnp.bfloat16)
        wwo[0] = wo_ref[HALF:, :].astype(jnp.bfloat16)

        rb = lax.broadcasted_iota(jnp.int32, (TS, TS), 0) // BLK
        cb = lax.broadcasted_iota(jnp.int32, (TS, TS), 1) // BLK
        mask_buf[...] = jnp.where(cb <= rb, 0.0, -1e9)

        out_ref[0] = jnp.zeros((SQ, D_MODEL), jnp.float32)
        xb = x_ref[0].astype(jnp.bfloat16)

        barrier = pltpu.get_barrier_semaphore()
        for nbr in (left, right):
            pl.semaphore_signal(
                barrier, inc=1,
                device_id=(nbr,), device_id_type=pl.DeviceIdType.MESH,
            )
        pl.semaphore_wait(barrier, 2)

        def chain(buf, ci, hh, dev):
            return pltpu.make_async_remote_copy(
                src_ref=buf.at[hh], dst_ref=buf.at[hh + 1],
                send_sem=ssem.at[ci, hh], recv_sem=rsem.at[ci, hh],
                device_id=(dev,), device_id_type=pl.DeviceIdType.MESH)

        def step(h, carry):
            par = lax.rem(h, 2)
            hm = lax.max(h - 1, 0)
            hc = lax.min(h, N_DEV - 2)

            @pl.when(h > 0)
            def _():
                chain(cwq, 0, hm, right).wait_recv()
                chain(wwq, 2, hm, left).wait_recv()

            @pl.when(h < N_DEV - 1)
            def _():
                chain(cwq, 0, hc, right).start()
                chain(wwq, 2, hc, left).start()

            @pl.when(h < N_DEV - 1)
            def _():
                for cp in kv_copies(h + 1, 1 - par):
                    cp.start()

            for cp in kv_copies(h, par):
                cp.wait()

            ck = (((1,), (1,)), ((), ()))
            cv = (((1,), (0,)), ((), ()))
            for qbuf, obuf, kbuf, vbuf, oci, dev in (
                (cwq, cwo, kcw, vcw, 1, right),
                (wwq, wwo, kww, vww, 3, left),
            ):
                wq_h = qbuf[h]
                q = lax.dot_general(
                    xb, wq_h, (((1,), (0,)), ((), ())),
                    preferred_element_type=jnp.float32)
                for i in range(HH):
                    for t in range(T):
                        r0 = t * TS
                        qt = q[r0:r0 + TS, i * DH:(i + 1) * DH]
                        sd = lax.dot_general(
                            qt, kbuf[par, i, r0:r0 + TS, :], ck,
                            preferred_element_type=jnp.float32)
                        wd = jnp.exp(sd * SCALE + mask_buf[...])
                        den = jnp.sum(wd, axis=1, keepdims=True)
                        ce = lax.dot_general(
                            wd, vbuf[par, i, r0:r0 + TS, :], cv,
                            preferred_element_type=jnp.float32)
                        if t > 0:
                            sf = lax.dot_general(
                                qt, kbuf[par, i, :r0, :], ck,
                                preferred_element_type=jnp.float32)
                            wf = jnp.exp(sf * SCALE)
                            den = den + jnp.sum(wf, axis=1, keepdims=True)
                            ce = ce + lax.dot_general(
                                wf, vbuf[par, i, :r0, :], cv,
                                preferred_element_type=jnp.float32)
                        ctx3[r0:r0 + TS, i * DH:(i + 1) * DH] = ce / den
                ctxt = ctx3[...].astype(jnp.bfloat16)

                @pl.when(h > 0)
                def _():
                    chain(obuf, oci, hm, dev).wait_recv()

                @pl.when(h < N_DEV - 1)
                def _():
                    chain(obuf, oci, hc, dev).start()

                contrib = lax.dot_general(
                    ctxt, obuf[h], (((1,), (0,)), ((), ())),
                    preferred_element_type=jnp.float32)
                out_ref[0] = out_ref[0] + contrib

            return carry

        lax.fori_loop(0, N_DEV, step, 0)

        for hh in range(N_DEV - 1):
            chain(cwq, 0, hh, right).wait_send()
            chain(cwo, 1, hh, right).wait_send()
            chain(wwq, 2, hh, left).wait_send()
            chain(wwo, 3, hh, left).wait_send()

    return pl.pallas_call(
        body,
        out_shape=jax.ShapeDtypeStruct((1, SQ, D_MODEL), jnp.float32),
        in_specs=[
            pl.BlockSpec(memory_space=pltpu.VMEM),
            pl.BlockSpec(memory_space=pltpu.VMEM),
            pl.BlockSpec(memory_space=pl.ANY),
            pl.BlockSpec(memory_space=pl.ANY),
            pl.BlockSpec(memory_space=pltpu.VMEM),
        ],
        out_specs=pl.BlockSpec(memory_space=pltpu.VMEM),
        scratch_shapes=[
            pltpu.VMEM((N_DEV, D_MODEL, HALF), jnp.bfloat16),
            pltpu.VMEM((N_DEV, HALF, D_MODEL), jnp.bfloat16),
            pltpu.VMEM((N_DEV, D_MODEL, HALF), jnp.bfloat16),
            pltpu.VMEM((N_DEV, HALF, D_MODEL), jnp.bfloat16),
            pltpu.VMEM((2, HH, SKV, DH), jnp.float32),
            pltpu.VMEM((2, HH, SKV, DH), jnp.float32),
            pltpu.VMEM((2, HH, SKV, DH), jnp.float32),
            pltpu.VMEM((2, HH, SKV, DH), jnp.float32),
            pltpu.VMEM((SQ, HALF), jnp.float32),
            pltpu.VMEM((TS, TS), jnp.float32),
            pltpu.SemaphoreType.DMA((4, N_DEV - 1)),
            pltpu.SemaphoreType.DMA((4, N_DEV - 1)),
            pltpu.SemaphoreType.DMA((2, 4 * HH)),
        ],
        compiler_params=pltpu.CompilerParams(
            collective_id=0,
            vmem_limit_bytes=58 * 1024 * 1024,
        ),
    )(x, Wq, K_ext, V_ext, Wo)
